# baseline (speedup 1.0000x reference)
import functools

import jax
import jax.numpy as jnp
from jax import lax
from jax.experimental import pallas as pl
from jax.experimental.pallas import tpu as pltpu
from jax.experimental.pallas import tpu_sc as plsc

_LANES = 16


def _pool_body(x_ref, out_ref):
    s = pl.program_id(1)

    @pl.when(s == 0)
    def _():
        out_ref[...] = jnp.zeros_like(out_ref)

    out_ref[...] += jnp.sum(x_ref[...], axis=-1, keepdims=True)


def _route_body(pooled_hbm, wg_hbm, bg_hbm, mask_hbm, pooled_v, wg_v, bg_v,
                mask_v, *, batch, feat, experts, inv_hw):
    cid = lax.axis_index("c")
    sid = lax.axis_index("s")
    is_lead = jnp.logical_and(cid == 0, sid == 0)

    @pl.when(is_lead)
    def _():
        pltpu.sync_copy(pooled_hbm, pooled_v)
        pltpu.sync_copy(wg_hbm, wg_v)
        pltpu.sync_copy(bg_hbm, bg_v)

        iot = jnp.arange(_LANES, dtype=jnp.int32)
        bgv = bg_v[...]
        nj = feat // _LANES
        for b in range(batch):
            lvec = jnp.full((_LANES,), -1e30, jnp.float32)
            for e in range(experts):
                acc = jnp.zeros((_LANES,), jnp.float32)
                for j in range(nj):
                    acc = acc + (pooled_v[b, pl.ds(j * _LANES, _LANES)] *
                                 wg_v[e, pl.ds(j * _LANES, _LANES)])
                s_e = jnp.sum(acc)
                lvec = jnp.where(iot == e, s_e * inv_hw, lvec)
            lvec = jnp.where(iot < experts, lvec + bgv, -1e30)
            mx = jnp.max(lvec)
            ex = jnp.exp(lvec - mx)
            w = ex / jnp.sum(ex)
            m1 = jnp.max(w)
            i1 = plsc.all_reduce_ffs(jnp.logical_and(w == m1, iot < experts))
            w2 = jnp.where(iot == i1, -1.0, w)
            m2 = jnp.max(w2)
            i2 = plsc.all_reduce_ffs(jnp.logical_and(w2 == m2, iot < experts))
            maskw = jnp.where(iot == i1, m1,
                              jnp.where(iot == i2, m2, 0.0))
            mask_v[b, :] = maskw
        pltpu.sync_copy(mask_v, mask_hbm)


def _apply_body(x_ref, mask_ref, we_ref, be_ref, k_ref, out_ref, weff_ref,
                bias_ref, *, experts):
    s = pl.program_id(1)

    @pl.when(s == 0)
    def _():
        m = mask_ref[0, 0, :experts]
        sig = 1.0 / (1.0 + jnp.exp(-k_ref[0, 0]))
        weff_ref[...] = (jnp.sum(we_ref[...] * m[:, None, None], axis=0)
                         * sig[:, None]).astype(jnp.bfloat16)
        bias_ref[...] = (jnp.sum(be_ref[...] * m[:, None], axis=0)
                         * sig)[None, :]

    xb = x_ref[0]
    y = jnp.dot(weff_ref[...], xb.astype(jnp.bfloat16),
                preferred_element_type=jnp.float32)
    out_ref[0] = xb + y + bias_ref[0, :][:, None]


def kernel(x, k, Wg, bg, We, be):
    B, C, H, W = x.shape
    E = Wg.shape[0]
    HW = H * W

    x3 = x.reshape(B, C, HW)
    k2 = k.reshape(B, 1, C)
    bg16 = jnp.zeros((_LANES,), bg.dtype).at[:E].set(bg)

    n_s = 8
    bs = HW // n_s
    pooled = pl.pallas_call(
        _pool_body,
        grid=(B, n_s),
        in_specs=[pl.BlockSpec((1, C, bs), lambda b, s: (b, 0, s))],
        out_specs=pl.BlockSpec((1, C, 1), lambda b, s: (b, 0, 0)),
        out_shape=jax.ShapeDtypeStruct((B, C, 1), jnp.float32),
    )(x3).reshape(B, C)

    mesh = plsc.VectorSubcoreMesh(core_axis_name="c", subcore_axis_name="s")
    route = functools.partial(
        pl.kernel,
        out_type=jax.ShapeDtypeStruct((B, _LANES), jnp.float32),
        mesh=mesh,
        scratch_types=[
            pltpu.VMEM((B, C), jnp.float32),
            pltpu.VMEM((E, C), jnp.float32),
            pltpu.VMEM((_LANES,), jnp.float32),
            pltpu.VMEM((B, _LANES), jnp.float32),
        ],
        compiler_params=pltpu.CompilerParams(needs_layout_passes=False),
    )(functools.partial(_route_body, batch=B, feat=C, experts=E,
                        inv_hw=1.0 / HW))
    mask = route(pooled, Wg, bg16).reshape(B, 1, _LANES)

    out3 = pl.pallas_call(
        functools.partial(_apply_body, experts=E),
        grid=(B, n_s),
        in_specs=[
            pl.BlockSpec((1, C, bs), lambda b, s: (b, 0, s)),
            pl.BlockSpec((1, 1, _LANES), lambda b, s: (b, 0, 0)),
            pl.BlockSpec((E, C, C), lambda b, s: (0, 0, 0)),
            pl.BlockSpec((E, C), lambda b, s: (0, 0)),
            pl.BlockSpec((1, 1, C), lambda b, s: (b, 0, 0)),
        ],
        out_specs=pl.BlockSpec((1, C, bs), lambda b, s: (b, 0, s)),
        out_shape=jax.ShapeDtypeStruct((B, C, HW), jnp.float32),
        scratch_shapes=[
            pltpu.VMEM((C, C), jnp.bfloat16),
            pltpu.VMEM((1, C), jnp.float32),
        ],
    )(x3, mask, We, be, k2)

    return out3.reshape(B, C, H, W)


# speedup vs baseline: 1.0586x; 1.0586x over previous
import functools

import jax
import jax.numpy as jnp
from jax import lax
from jax.experimental import pallas as pl
from jax.experimental.pallas import tpu as pltpu
from jax.experimental.pallas import tpu_sc as plsc

_LANES = 16


def _pool_body(x_ref, out_ref):
    s = pl.program_id(1)

    @pl.when(s == 0)
    def _():
        out_ref[...] = jnp.zeros_like(out_ref)

    out_ref[...] += jnp.sum(x_ref[...], axis=-1, keepdims=True)


def _route_body(pooled_hbm, wg_hbm, bg_hbm, mask_hbm, pooled_v, wg_v, bg_v,
                mask_v, *, batch, feat, experts, inv_hw):
    cid = lax.axis_index("c")
    sid = lax.axis_index("s")
    is_lead = jnp.logical_and(cid == 0, sid == 0)

    @pl.when(is_lead)
    def _():
        pltpu.sync_copy(pooled_hbm, pooled_v)
        pltpu.sync_copy(wg_hbm, wg_v)
        pltpu.sync_copy(bg_hbm, bg_v)

        iot = jnp.arange(_LANES, dtype=jnp.int32)
        bgv = bg_v[...]
        nj = feat // _LANES
        for b in range(batch):
            lvec = jnp.full((_LANES,), -1e30, jnp.float32)
            for e in range(experts):
                acc = jnp.zeros((_LANES,), jnp.float32)
                for j in range(nj):
                    acc = acc + (pooled_v[b, pl.ds(j * _LANES, _LANES)] *
                                 wg_v[e, pl.ds(j * _LANES, _LANES)])
                s_e = jnp.sum(acc)
                lvec = jnp.where(iot == e, s_e * inv_hw, lvec)
            lvec = jnp.where(iot < experts, lvec + bgv, -1e30)
            mx = jnp.max(lvec)
            ex = jnp.exp(lvec - mx)
            w = ex / jnp.sum(ex)
            m1 = jnp.max(w)
            i1 = plsc.all_reduce_ffs(jnp.logical_and(w == m1, iot < experts))
            w2 = jnp.where(iot == i1, -1.0, w)
            m2 = jnp.max(w2)
            i2 = plsc.all_reduce_ffs(jnp.logical_and(w2 == m2, iot < experts))
            maskw = jnp.where(iot == i1, m1,
                              jnp.where(iot == i2, m2, 0.0))
            mask_v[b, :] = maskw
        pltpu.sync_copy(mask_v, mask_hbm)


def _apply_body(x_ref, mask_ref, we_ref, be_ref, k_ref, out_ref, weff_ref,
                bias_ref, *, experts):
    s = pl.program_id(1)

    @pl.when(s == 0)
    def _():
        m = mask_ref[0, 0, :experts]
        sig = 1.0 / (1.0 + jnp.exp(-k_ref[0, 0]))
        weff_ref[...] = (jnp.sum(we_ref[...] * m[:, None, None], axis=0)
                         * sig[:, None]).astype(jnp.bfloat16)
        bias_ref[...] = (jnp.sum(be_ref[...] * m[:, None], axis=0)
                         * sig)[None, :]

    xb = x_ref[0]
    y = jnp.dot(weff_ref[...], xb.astype(jnp.bfloat16),
                preferred_element_type=jnp.float32)
    out_ref[0] = xb + y + bias_ref[0, :][:, None]


def kernel(x, k, Wg, bg, We, be):
    B, C, H, W = x.shape
    E = Wg.shape[0]
    HW = H * W

    x3 = x.reshape(B, C, HW)
    k2 = k.reshape(B, 1, C)
    bg16 = jnp.zeros((_LANES,), bg.dtype).at[:E].set(bg)

    n_s = 2
    bs = HW // n_s
    pooled = pl.pallas_call(
        _pool_body,
        grid=(B, n_s),
        in_specs=[pl.BlockSpec((1, C, bs), lambda b, s: (b, 0, s))],
        out_specs=pl.BlockSpec((1, C, 1), lambda b, s: (b, 0, 0)),
        out_shape=jax.ShapeDtypeStruct((B, C, 1), jnp.float32),
    )(x3).reshape(B, C)

    mesh = plsc.VectorSubcoreMesh(core_axis_name="c", subcore_axis_name="s")
    route = functools.partial(
        pl.kernel,
        out_type=jax.ShapeDtypeStruct((B, _LANES), jnp.float32),
        mesh=mesh,
        scratch_types=[
            pltpu.VMEM((B, C), jnp.float32),
            pltpu.VMEM((E, C), jnp.float32),
            pltpu.VMEM((_LANES,), jnp.float32),
            pltpu.VMEM((B, _LANES), jnp.float32),
        ],
        compiler_params=pltpu.CompilerParams(needs_layout_passes=False),
    )(functools.partial(_route_body, batch=B, feat=C, experts=E,
                        inv_hw=1.0 / HW))
    mask = route(pooled, Wg, bg16).reshape(B, 1, _LANES)

    out3 = pl.pallas_call(
        functools.partial(_apply_body, experts=E),
        grid=(B, n_s),
        in_specs=[
            pl.BlockSpec((1, C, bs), lambda b, s: (b, 0, s)),
            pl.BlockSpec((1, 1, _LANES), lambda b, s: (b, 0, 0)),
            pl.BlockSpec((E, C, C), lambda b, s: (0, 0, 0)),
            pl.BlockSpec((E, C), lambda b, s: (0, 0)),
            pl.BlockSpec((1, 1, C), lambda b, s: (b, 0, 0)),
        ],
        out_specs=pl.BlockSpec((1, C, bs), lambda b, s: (b, 0, s)),
        out_shape=jax.ShapeDtypeStruct((B, C, HW), jnp.float32),
        scratch_shapes=[
            pltpu.VMEM((C, C), jnp.bfloat16),
            pltpu.VMEM((1, C), jnp.float32),
        ],
    )(x3, mask, We, be, k2)

    return out3.reshape(B, C, H, W)
